# Initial kernel scaffold; baseline (speedup 1.0000x reference)
import jax
import jax.numpy as jnp
from jax.experimental import pallas as pl


def kernel(input_values, proj_weight, code_book):
    raise NotImplementedError("write your pallas kernel here")


# speedup vs baseline: 2.0443x; 2.0443x over previous
import functools

import jax
import jax.numpy as jnp
from jax import lax
from jax.experimental import pallas as pl

B, L, D = 4, 2048, 512
K, C = 8192, 16
TOK = 512
KT = 1024
N_TILES = (B * L) // TOK

SUM_ORDER = "strided"


def _sum_sq(sq):
    if SUM_ORDER == "seq":
        acc = sq[0]
        for c in range(1, C):
            acc = acc + sq[c]
        return acc
    if SUM_ORDER == "adjacent":
        while len(sq) > 1:
            sq = [sq[i] + sq[i + 1] for i in range(0, len(sq), 2)]
        return sq[0]
    while len(sq) > 1:
        h = len(sq) // 2
        sq = [sq[i] + sq[i + h] for i in range(h)]
    return sq[0]


def _vq_body(x_ref, w_ref, cbt_ref, out_ref):
    x = x_ref[...]
    w = w_ref[...]
    cbt = cbt_ref[...]
    t = lax.dot_general(x, w, (((1,), (1,)), ((), ())),
                        preferred_element_type=jnp.float32)

    best_v = jnp.full((TOK,), jnp.inf, jnp.float32)
    best_i = jnp.zeros((TOK,), jnp.int32)
    for j in range(K // KT):
        sq = []
        for c in range(C):
            dlt = t[:, c:c + 1] - cbt[c:c + 1, j * KT:(j + 1) * KT]
            sq.append(dlt * dlt)
        dist = jnp.sqrt(_sum_sq(sq))
        m = jnp.min(dist, axis=1)
        ii = lax.broadcasted_iota(jnp.int32, (TOK, KT), 1) + (j * KT)
        cand = jnp.min(jnp.where(dist == m[:, None], ii, jnp.int32(K)), axis=1)
        upd = m < best_v
        best_i = jnp.where(upd, cand, best_i)
        best_v = jnp.where(upd, m, best_v)
    out_ref[0, 0, :] = best_i


@functools.partial(jax.jit, static_argnums=())
def kernel(input_values, proj_weight, code_book):
    x = input_values.reshape(B * L, D)
    cbt = code_book.T
    out = pl.pallas_call(
        _vq_body,
        grid=(N_TILES,),
        in_specs=[
            pl.BlockSpec((TOK, D), lambda i: (i, 0)),
            pl.BlockSpec((C, D), lambda i: (0, 0)),
            pl.BlockSpec((C, K), lambda i: (0, 0)),
        ],
        out_specs=pl.BlockSpec((1, 1, TOK), lambda i: (i, 0, 0)),
        out_shape=jax.ShapeDtypeStruct((N_TILES, 1, TOK), jnp.int32),
    )(x, proj_weight, cbt)
    return out.reshape(B, L)
